# baseline (speedup 1.0000x reference)
import functools

import jax
import jax.numpy as jnp
from jax import lax
from jax.experimental import pallas as pl
from jax.experimental.pallas import tpu as pltpu
from jax.experimental.pallas import tpu_sc as plsc

B = 16384
D_EMB = 64
N_IND_SLOTS = 20
IND_SZ = 16
FTYPE_SZ = 4
N_INDS = 1000
N_FTYPES = 16
AF_ROWS = IND_SZ + FTYPE_SZ

NC = 2
NS = 16
L = 16
NW = NC * NS
BPW = B // NW
CHUNK = 128
NCHUNK = BPW // CHUNK
NGROUP = BPW // L


def _sc_body(funds_h, startups_h, ind_t_h, ftype_h, user_h, item_h,
             itblT_h, ftblT_h,
             ui_out, af_out,
             *refs):
  idxu_v = refs[0:NCHUNK]
  idxi_v = refs[NCHUNK:2 * NCHUNK]
  (paru_v, pari_v, ftidx_v, indv_v, itblT_v, ftblT_v,
   urows0, urows1, irows0, irows1, ex_v, af_stage, sem) = refs[2 * NCHUNK:]
  urows = (urows0, urows1)
  irows = (irows0, irows1)
  wid = lax.axis_index("s") * NC + lax.axis_index("c")
  base = wid * BPW

  iota = lax.iota(jnp.int32, L)
  onev = jnp.full((L,), 1, jnp.int32)

  for k in range(NCHUNK):
    pltpu.sync_copy(funds_h.at[pl.ds(base + k * CHUNK, CHUNK)], idxu_v[k])
    pltpu.sync_copy(startups_h.at[pl.ds(base + k * CHUNK, CHUNK)], idxi_v[k])
  for k in range(NCHUNK):
    for q in range(CHUNK // L):
      s = q * L
      vu = idxu_v[k][pl.ds(s, L)]
      vi = idxi_v[k][pl.ds(s, L)]
      paru_v[pl.ds(k * CHUNK + s, L)] = jnp.bitwise_and(vu, onev)
      pari_v[pl.ds(k * CHUNK + s, L)] = jnp.bitwise_and(vi, onev)
      idxu_v[k][pl.ds(s, L)] = lax.shift_right_logical(vu, onev)
      idxi_v[k][pl.ds(s, L)] = lax.shift_right_logical(vi, onev)

  def fire(p):
    return [
        pltpu.async_copy(user_h.at[idxu_v[p]], urows[p % 2], sem),
        pltpu.async_copy(item_h.at[idxi_v[p]], irows[p % 2], sem),
    ]

  copies = fire(0)

  pltpu.sync_copy(itblT_h, itblT_v)
  pltpu.sync_copy(ftblT_h, ftblT_v)
  pltpu.sync_copy(ftype_h.at[pl.ds(base, BPW)], ftidx_v)
  for j in range(N_IND_SLOTS):
    pltpu.sync_copy(ind_t_h.at[j, pl.ds(base, BPW)], indv_v.at[j])

  def group(g, carry):
    s = pl.multiple_of(g * L, L)
    idxs = [indv_v[j, pl.ds(s, L)] for j in range(N_IND_SLOTS)]
    cnt = jnp.full((L,), 0.0, jnp.float32)
    for j in range(N_IND_SLOTS):
      cnt = cnt + jnp.minimum(idxs[j], 1).astype(jnp.float32)
    inv = jnp.full((L,), 1.0, jnp.float32) / cnt
    for c in range(IND_SZ):
      colv = jnp.full((L,), c, jnp.int32)
      acc = plsc.load_gather(itblT_v, [colv, idxs[0]])
      for j in range(1, N_IND_SLOTS):
        acc = acc + plsc.load_gather(itblT_v, [colv, idxs[j]])
      af_stage[c, pl.ds(s, L)] = acc * inv
    ftv = ftidx_v[pl.ds(s, L)]
    for c in range(FTYPE_SZ):
      colv = jnp.full((L,), c, jnp.int32)
      af_stage[IND_SZ + c, pl.ds(s, L)] = plsc.load_gather(ftblT_v, [colv, ftv])
    return carry

  lax.fori_loop(0, NGROUP, group, 0)
  pltpu.sync_copy(af_stage, af_out.at[:, pl.ds(base, BPW)])

  for p in range(NCHUNK):
    for c in copies:
      c.wait()
    if p + 1 < NCHUNK:
      copies = fire(p + 1)
    ub, ib = urows[p % 2], irows[p % 2]

    def xrow(r, carry, p=p, ub=ub, ib=ib):
      rsplat = jnp.full((L,), r, jnp.int32)
      pu = plsc.load_gather(paru_v, [rsplat + (p * CHUNK)]) * D_EMB
      pi = plsc.load_gather(pari_v, [rsplat + (p * CHUNK)]) * D_EMB
      for c4 in range(D_EMB // L):
        fvec = iota + (c4 * L)
        vu = plsc.load_gather(ub, [rsplat, pu + fvec])
        vi = plsc.load_gather(ib, [rsplat, pi + fvec])
        plsc.store_scatter(ex_v, [rsplat, fvec], vu)
        plsc.store_scatter(ex_v, [rsplat, fvec + D_EMB], vi)
      return carry

    lax.fori_loop(0, CHUNK, xrow, 0)
    pltpu.sync_copy(ex_v, ui_out.at[pl.ds(base + p * CHUNK, CHUNK)])


def _sc_gather(funds, startups, industries_t, funding_type,
               user2, item2, itblT, ftblT):
  mesh = plsc.VectorSubcoreMesh(
      core_axis_name="c", subcore_axis_name="s",
      num_cores=NC, num_subcores=NS)
  f32 = jnp.float32
  out_type = (
      jax.ShapeDtypeStruct((B, 2 * D_EMB), f32),
      jax.ShapeDtypeStruct((AF_ROWS, B), f32),
  )
  scratch = [
      *[pltpu.VMEM((CHUNK,), jnp.int32) for _ in range(NCHUNK)],
      *[pltpu.VMEM((CHUNK,), jnp.int32) for _ in range(NCHUNK)],
      pltpu.VMEM((BPW,), jnp.int32),
      pltpu.VMEM((BPW,), jnp.int32),
      pltpu.VMEM((BPW,), jnp.int32),
      pltpu.VMEM((N_IND_SLOTS, BPW), jnp.int32),
      pltpu.VMEM((IND_SZ, N_INDS), f32),
      pltpu.VMEM((FTYPE_SZ, N_FTYPES), f32),
      pltpu.VMEM((CHUNK, 2 * D_EMB), f32),
      pltpu.VMEM((CHUNK, 2 * D_EMB), f32),
      pltpu.VMEM((CHUNK, 2 * D_EMB), f32),
      pltpu.VMEM((CHUNK, 2 * D_EMB), f32),
      pltpu.VMEM((CHUNK, 2 * D_EMB), f32),
      pltpu.VMEM((AF_ROWS, BPW), f32),
      pltpu.SemaphoreType.DMA,
  ]
  run = pl.kernel(_sc_body, out_type=out_type, mesh=mesh,
                  scratch_types=scratch,
                  compiler_params=pltpu.CompilerParams(
                      use_tc_tiling_on_sc=False,
                      needs_layout_passes=False))
  return run(funds, startups, industries_t, funding_type,
             user2, item2, itblT, ftblT)


BT = 2048


def _mlp_body(ui_ref, af_ref,
              w1u_ref, w1i_ref, w1af_ref, b1_ref,
              w2_ref, b2_ref, w3_ref, b3_ref, out_ref):
  dot = functools.partial(jnp.dot, preferred_element_type=jnp.float32)
  u = ui_ref[:, :D_EMB]
  it = ui_ref[:, D_EMB:]
  x = (dot(u, w1u_ref[:]) + dot(it, w1i_ref[:])
       + lax.dot_general(af_ref[:], w1af_ref[:], (((0,), (0,)), ((), ())),
                         preferred_element_type=jnp.float32)
       + b1_ref[:])
  h1 = jnp.maximum(x, 0.0)
  h2 = jnp.maximum(dot(h1, w2_ref[:]) + b2_ref[:], 0.0)
  out_ref[:] = jnp.maximum(dot(h2, w3_ref[:]) + b3_ref[:], 0.0)


def _mlp(ui, afT, W1u, W1i, W1af, b1, W2, b2, W3, b3):
  h1, h2, d_out = W2.shape[0], W3.shape[0], W3.shape[1]
  grid = (B // BT,)
  full = lambda shape: pl.BlockSpec(shape, lambda i: (0, 0))
  return pl.pallas_call(
      _mlp_body,
      grid=grid,
      in_specs=[
          pl.BlockSpec((BT, 2 * D_EMB), lambda i: (i, 0)),
          pl.BlockSpec((AF_ROWS, BT), lambda i: (0, i)),
          full(W1u.shape), full(W1i.shape), full(W1af.shape),
          full((1, h1)),
          full(W2.shape), full((1, h2)),
          full(W3.shape), full((1, d_out)),
      ],
      out_specs=pl.BlockSpec((BT, d_out), lambda i: (i, 0)),
      out_shape=jax.ShapeDtypeStruct((B, d_out), jnp.float32),
  )(ui, afT, W1u, W1i, W1af, b1.reshape(1, -1),
    W2, b2.reshape(1, -1), W3, b3.reshape(1, -1))


def kernel(funds, startups, industries, funding_type, user_table, item_table,
           ind_table, ftype_table, W1, b1, W2, b2, W3, b3):
  funds = funds.astype(jnp.int32)
  startups = startups.astype(jnp.int32)
  funding_type = funding_type.astype(jnp.int32)
  industries_t = industries.astype(jnp.int32).T
  user2 = user_table.reshape(-1, 2 * D_EMB)
  item2 = item_table.reshape(-1, 2 * D_EMB)
  itblT = ind_table.T.at[:, 0].set(0.0)
  ftblT = ftype_table.T
  ui, afT = _sc_gather(funds, startups, industries_t, funding_type,
                       user2, item2, itblT, ftblT)
  W1u = W1[:D_EMB]
  W1i = W1[D_EMB:2 * D_EMB]
  W1af = jnp.concatenate([W1[2 * D_EMB + FTYPE_SZ:], W1[2 * D_EMB:2 * D_EMB + FTYPE_SZ]], axis=0)
  return _mlp(ui, afT, W1u, W1i, W1af, b1, W2, b2, W3, b3)


# speedup vs baseline: 1.0505x; 1.0505x over previous
import functools

import jax
import jax.numpy as jnp
from jax import lax
from jax.experimental import pallas as pl
from jax.experimental.pallas import tpu as pltpu
from jax.experimental.pallas import tpu_sc as plsc

B = 16384
D_EMB = 64
N_IND_SLOTS = 20
IND_SZ = 16
FTYPE_SZ = 4
N_INDS = 1000
N_FTYPES = 16
AF_ROWS = IND_SZ + FTYPE_SZ

NC = 2
NS = 16
L = 16
NW = NC * NS
BPW = B // NW
CHUNK = 128
NCHUNK = BPW // CHUNK
NGROUP = BPW // L


def _sc_ui_body(funds_h, startups_h, user_h, item_h, ui_out, *refs):
  idxo_u, idxo_i = refs[0], refs[1]
  idxs_u = refs[2:2 + NCHUNK]
  idxs_i = refs[2 + NCHUNK:2 + 2 * NCHUNK]
  urows0, urows1, irows0, irows1, ex_v, sem = refs[2 + 2 * NCHUNK:]
  urows = (urows0, urows1)
  irows = (irows0, irows1)
  wid = lax.axis_index("s") * NC + lax.axis_index("c")
  base = wid * BPW

  iota = lax.iota(jnp.int32, L)
  onev = jnp.full((L,), 1, jnp.int32)

  pltpu.sync_copy(funds_h.at[pl.ds(base, BPW)], idxo_u)
  pltpu.sync_copy(startups_h.at[pl.ds(base, BPW)], idxo_i)

  for k in range(NCHUNK):
    for q in range(CHUNK // L):
      lanes = iota + (k * CHUNK + q * L)
      dst = iota + (q * L)
      vu = plsc.load_gather(idxo_u, [lanes])
      vi = plsc.load_gather(idxo_i, [lanes])
      plsc.store_scatter(idxs_u[k], [dst], lax.shift_right_logical(vu, onev))
      plsc.store_scatter(idxs_i[k], [dst], lax.shift_right_logical(vi, onev))

  def fire(p):
    return [
        pltpu.async_copy(user_h.at[idxs_u[p]], urows[p % 2], sem),
        pltpu.async_copy(item_h.at[idxs_i[p]], irows[p % 2], sem),
    ]

  copies = fire(0)

  for p in range(NCHUNK):
    for c in copies:
      c.wait()
    if p + 1 < NCHUNK:
      copies = fire(p + 1)
    ub, ib = urows[p % 2], irows[p % 2]

    def xrow(r, carry, p=p, ub=ub, ib=ib):
      rsplat = jnp.full((L,), r, jnp.int32)
      pu = jnp.bitwise_and(plsc.load_gather(idxo_u, [rsplat + (p * CHUNK)]),
                           onev) * D_EMB
      pi = jnp.bitwise_and(plsc.load_gather(idxo_i, [rsplat + (p * CHUNK)]),
                           onev) * D_EMB
      for c4 in range(D_EMB // L):
        fvec = iota + (c4 * L)
        vu = plsc.load_gather(ub, [rsplat, pu + fvec])
        vi = plsc.load_gather(ib, [rsplat, pi + fvec])
        plsc.store_scatter(ex_v, [rsplat, fvec], vu)
        plsc.store_scatter(ex_v, [rsplat, fvec + D_EMB], vi)
      return carry

    lax.fori_loop(0, CHUNK, xrow, 0)
    pltpu.sync_copy(ex_v, ui_out.at[pl.ds(base + p * CHUNK, CHUNK)])


def _sc_ui(funds, startups, user2, item2):
  mesh = plsc.VectorSubcoreMesh(
      core_axis_name="c", subcore_axis_name="s",
      num_cores=NC, num_subcores=NS)
  f32 = jnp.float32
  scratch = [
      pltpu.VMEM((BPW,), jnp.int32),
      pltpu.VMEM((BPW,), jnp.int32),
      *[pltpu.VMEM((CHUNK,), jnp.int32) for _ in range(NCHUNK)],
      *[pltpu.VMEM((CHUNK,), jnp.int32) for _ in range(NCHUNK)],
      pltpu.VMEM((CHUNK, 2 * D_EMB), f32),
      pltpu.VMEM((CHUNK, 2 * D_EMB), f32),
      pltpu.VMEM((CHUNK, 2 * D_EMB), f32),
      pltpu.VMEM((CHUNK, 2 * D_EMB), f32),
      pltpu.VMEM((CHUNK, 2 * D_EMB), f32),
      pltpu.SemaphoreType.DMA,
  ]
  run = pl.kernel(_sc_ui_body,
                  out_type=jax.ShapeDtypeStruct((B, 2 * D_EMB), f32),
                  mesh=mesh, scratch_types=scratch,
                  compiler_params=pltpu.CompilerParams(
                      needs_layout_passes=False))
  return run(funds, startups, user2, item2)


def _sc_af_body(ind_t_h, ftype_h, itblT_h, ftblT_h, af_out,
                ftidx_v, indv_v, itblT_v, ftblT_v, af_stage):
  wid = lax.axis_index("s") * NC + lax.axis_index("c")
  base = wid * BPW

  pltpu.sync_copy(itblT_h, itblT_v)
  pltpu.sync_copy(ftblT_h, ftblT_v)
  pltpu.sync_copy(ftype_h.at[pl.ds(base, BPW)], ftidx_v)
  for j in range(N_IND_SLOTS):
    pltpu.sync_copy(ind_t_h.at[j, pl.ds(base, BPW)], indv_v.at[j])

  def group(g, carry):
    s = pl.multiple_of(g * L, L)
    idxs = [indv_v[j, pl.ds(s, L)] for j in range(N_IND_SLOTS)]
    cnt = jnp.full((L,), 0.0, jnp.float32)
    for j in range(N_IND_SLOTS):
      cnt = cnt + jnp.minimum(idxs[j], 1).astype(jnp.float32)
    inv = jnp.full((L,), 1.0, jnp.float32) / cnt
    for c in range(IND_SZ):
      colv = jnp.full((L,), c, jnp.int32)
      acc = plsc.load_gather(itblT_v, [colv, idxs[0]])
      for j in range(1, N_IND_SLOTS):
        acc = acc + plsc.load_gather(itblT_v, [colv, idxs[j]])
      af_stage[c, pl.ds(s, L)] = acc * inv
    ftv = ftidx_v[pl.ds(s, L)]
    for c in range(FTYPE_SZ):
      colv = jnp.full((L,), c, jnp.int32)
      af_stage[IND_SZ + c, pl.ds(s, L)] = plsc.load_gather(ftblT_v, [colv, ftv])
    return carry

  lax.fori_loop(0, NGROUP, group, 0)
  pltpu.sync_copy(af_stage, af_out.at[:, pl.ds(base, BPW)])


def _sc_af(industries_t, funding_type, itblT, ftblT):
  mesh = plsc.VectorSubcoreMesh(
      core_axis_name="c", subcore_axis_name="s",
      num_cores=NC, num_subcores=NS)
  f32 = jnp.float32
  scratch = [
      pltpu.VMEM((BPW,), jnp.int32),
      pltpu.VMEM((N_IND_SLOTS, BPW), jnp.int32),
      pltpu.VMEM((IND_SZ, N_INDS), f32),
      pltpu.VMEM((FTYPE_SZ, N_FTYPES), f32),
      pltpu.VMEM((AF_ROWS, BPW), f32),
  ]
  run = pl.kernel(_sc_af_body,
                  out_type=jax.ShapeDtypeStruct((AF_ROWS, B), f32),
                  mesh=mesh, scratch_types=scratch,
                  compiler_params=pltpu.CompilerParams(
                      use_tc_tiling_on_sc=False,
                      needs_layout_passes=False))
  return run(industries_t, funding_type, itblT, ftblT)


BT = 2048


def _mlp_body(ui_ref, af_ref,
              w1u_ref, w1i_ref, w1af_ref, b1_ref,
              w2_ref, b2_ref, w3_ref, b3_ref, out_ref):
  dot = functools.partial(jnp.dot, preferred_element_type=jnp.float32)
  u = ui_ref[:, :D_EMB]
  it = ui_ref[:, D_EMB:]
  x = (dot(u, w1u_ref[:]) + dot(it, w1i_ref[:])
       + lax.dot_general(af_ref[:], w1af_ref[:], (((0,), (0,)), ((), ())),
                         preferred_element_type=jnp.float32)
       + b1_ref[:])
  h1 = jnp.maximum(x, 0.0)
  h2 = jnp.maximum(dot(h1, w2_ref[:]) + b2_ref[:], 0.0)
  out_ref[:] = jnp.maximum(dot(h2, w3_ref[:]) + b3_ref[:], 0.0)


def _mlp(ui, afT, W1u, W1i, W1af, b1, W2, b2, W3, b3):
  h1, h2, d_out = W2.shape[0], W3.shape[0], W3.shape[1]
  grid = (B // BT,)
  full = lambda shape: pl.BlockSpec(shape, lambda i: (0, 0))
  return pl.pallas_call(
      _mlp_body,
      grid=grid,
      in_specs=[
          pl.BlockSpec((BT, 2 * D_EMB), lambda i: (i, 0)),
          pl.BlockSpec((AF_ROWS, BT), lambda i: (0, i)),
          full(W1u.shape), full(W1i.shape), full(W1af.shape),
          full((1, h1)),
          full(W2.shape), full((1, h2)),
          full(W3.shape), full((1, d_out)),
      ],
      out_specs=pl.BlockSpec((BT, d_out), lambda i: (i, 0)),
      out_shape=jax.ShapeDtypeStruct((B, d_out), jnp.float32),
  )(ui, afT, W1u, W1i, W1af, b1.reshape(1, -1),
    W2, b2.reshape(1, -1), W3, b3.reshape(1, -1))


def kernel(funds, startups, industries, funding_type, user_table, item_table,
           ind_table, ftype_table, W1, b1, W2, b2, W3, b3):
  funds = funds.astype(jnp.int32)
  startups = startups.astype(jnp.int32)
  funding_type = funding_type.astype(jnp.int32)
  industries_t = industries.astype(jnp.int32).T
  user2 = user_table.reshape(-1, 2 * D_EMB)
  item2 = item_table.reshape(-1, 2 * D_EMB)
  itblT = ind_table.T.at[:, 0].set(0.0)
  ftblT = ftype_table.T
  ui = _sc_ui(funds, startups, user2, item2)
  afT = _sc_af(industries_t, funding_type, itblT, ftblT)
  W1u = W1[:D_EMB]
  W1i = W1[D_EMB:2 * D_EMB]
  W1af = jnp.concatenate([W1[2 * D_EMB + FTYPE_SZ:], W1[2 * D_EMB:2 * D_EMB + FTYPE_SZ]], axis=0)
  return _mlp(ui, afT, W1u, W1i, W1af, b1, W2, b2, W3, b3)


# speedup vs baseline: 1.1342x; 1.0796x over previous
import functools

import jax
import jax.numpy as jnp
from jax import lax
from jax.experimental import pallas as pl
from jax.experimental.pallas import tpu as pltpu
from jax.experimental.pallas import tpu_sc as plsc

B = 16384
D_EMB = 64
N_IND_SLOTS = 20
IND_SZ = 16
FTYPE_SZ = 4
N_INDS = 1000
N_FTYPES = 16
AF_ROWS = IND_SZ + FTYPE_SZ

NC = 2
NS = 16
L = 16
NW = NC * NS
BPW = B // NW
CHUNK = 128
NCHUNK = BPW // CHUNK
NGROUP = BPW // L


def _sc_ui_body(funds_h, startups_h, user_h, item_h, ui_out, *refs):
  idxo_u, idxo_i = refs[0], refs[1]
  idxs_u = refs[2:2 + NCHUNK]
  idxs_i = refs[2 + NCHUNK:2 + 2 * NCHUNK]
  urows0, urows1, irows0, irows1, ex_v, sem = refs[2 + 2 * NCHUNK:]
  urows = (urows0, urows1)
  irows = (irows0, irows1)
  wid = lax.axis_index("s") * NC + lax.axis_index("c")
  base = wid * BPW

  iota = lax.iota(jnp.int32, L)
  onev = jnp.full((L,), 1, jnp.int32)

  pltpu.sync_copy(funds_h.at[pl.ds(base, BPW)], idxo_u)
  pltpu.sync_copy(startups_h.at[pl.ds(base, BPW)], idxo_i)

  for k in range(NCHUNK):
    for q in range(CHUNK // L):
      lanes = iota + (k * CHUNK + q * L)
      dst = iota + (q * L)
      vu = plsc.load_gather(idxo_u, [lanes])
      vi = plsc.load_gather(idxo_i, [lanes])
      plsc.store_scatter(idxs_u[k], [dst], vu)
      plsc.store_scatter(idxs_i[k], [dst], vi)

  def fire(p):
    return [
        pltpu.async_copy(user_h.at[idxs_u[p]], urows[p % 2], sem),
        pltpu.async_copy(item_h.at[idxs_i[p]], irows[p % 2], sem),
    ]

  copies = fire(0)

  for p in range(NCHUNK):
    for c in copies:
      c.wait()
    if p + 1 < NCHUNK:
      copies = fire(p + 1)
    ub, ib = urows[p % 2], irows[p % 2]

    def addrow(r, carry, ub=ub, ib=ib):
      rsplat = jnp.full((L,), r, jnp.int32)
      for c8 in range(2 * D_EMB // L):
        fvec = lax.iota(jnp.int32, L) + (c8 * L)
        vu = plsc.load_gather(ub, [rsplat, fvec])
        vi = plsc.load_gather(ib, [rsplat, fvec])
        plsc.store_scatter(ex_v, [rsplat, fvec], vu + vi)
      return carry

    lax.fori_loop(0, CHUNK, addrow, 0)
    pltpu.sync_copy(ex_v, ui_out.at[pl.ds(base + p * CHUNK, CHUNK)])


def _sc_ui(funds, startups, user2, item2):
  mesh = plsc.VectorSubcoreMesh(
      core_axis_name="c", subcore_axis_name="s",
      num_cores=NC, num_subcores=NS)
  f32 = jnp.float32
  scratch = [
      pltpu.VMEM((BPW,), jnp.int32),
      pltpu.VMEM((BPW,), jnp.int32),
      *[pltpu.VMEM((CHUNK,), jnp.int32) for _ in range(NCHUNK)],
      *[pltpu.VMEM((CHUNK,), jnp.int32) for _ in range(NCHUNK)],
      pltpu.VMEM((CHUNK, 2 * D_EMB), f32),
      pltpu.VMEM((CHUNK, 2 * D_EMB), f32),
      pltpu.VMEM((CHUNK, 2 * D_EMB), f32),
      pltpu.VMEM((CHUNK, 2 * D_EMB), f32),
      pltpu.VMEM((CHUNK, 2 * D_EMB), f32),
      pltpu.SemaphoreType.DMA,
  ]
  run = pl.kernel(_sc_ui_body,
                  out_type=jax.ShapeDtypeStruct((B, 2 * D_EMB), f32),
                  mesh=mesh, scratch_types=scratch,
                  compiler_params=pltpu.CompilerParams(
                      needs_layout_passes=False))
  return run(funds, startups, user2, item2)


def _sc_af_body(ind_t_h, ftype_h, itblT_h, ftblT_h, af_out,
                ftidx_v, indv_v, itblT_v, ftblT_v, af_stage):
  wid = lax.axis_index("s") * NC + lax.axis_index("c")
  base = wid * BPW

  pltpu.sync_copy(itblT_h, itblT_v)
  pltpu.sync_copy(ftblT_h, ftblT_v)
  pltpu.sync_copy(ftype_h.at[pl.ds(base, BPW)], ftidx_v)
  for j in range(N_IND_SLOTS):
    pltpu.sync_copy(ind_t_h.at[j, pl.ds(base, BPW)], indv_v.at[j])

  def group(g, carry):
    s = pl.multiple_of(g * L, L)
    idxs = [indv_v[j, pl.ds(s, L)] for j in range(N_IND_SLOTS)]
    cnt = jnp.full((L,), 0.0, jnp.float32)
    for j in range(N_IND_SLOTS):
      cnt = cnt + jnp.minimum(idxs[j], 1).astype(jnp.float32)
    inv = jnp.full((L,), 1.0, jnp.float32) / cnt
    for c in range(IND_SZ):
      colv = jnp.full((L,), c, jnp.int32)
      acc = plsc.load_gather(itblT_v, [colv, idxs[0]])
      for j in range(1, N_IND_SLOTS):
        acc = acc + plsc.load_gather(itblT_v, [colv, idxs[j]])
      af_stage[c, pl.ds(s, L)] = acc * inv
    ftv = ftidx_v[pl.ds(s, L)]
    for c in range(FTYPE_SZ):
      colv = jnp.full((L,), c, jnp.int32)
      af_stage[IND_SZ + c, pl.ds(s, L)] = plsc.load_gather(ftblT_v, [colv, ftv])
    return carry

  lax.fori_loop(0, NGROUP, group, 0)
  pltpu.sync_copy(af_stage, af_out.at[:, pl.ds(base, BPW)])


def _sc_af(industries_t, funding_type, itblT, ftblT):
  mesh = plsc.VectorSubcoreMesh(
      core_axis_name="c", subcore_axis_name="s",
      num_cores=NC, num_subcores=NS)
  f32 = jnp.float32
  scratch = [
      pltpu.VMEM((BPW,), jnp.int32),
      pltpu.VMEM((N_IND_SLOTS, BPW), jnp.int32),
      pltpu.VMEM((IND_SZ, N_INDS), f32),
      pltpu.VMEM((FTYPE_SZ, N_FTYPES), f32),
      pltpu.VMEM((AF_ROWS, BPW), f32),
  ]
  run = pl.kernel(_sc_af_body,
                  out_type=jax.ShapeDtypeStruct((AF_ROWS, B), f32),
                  mesh=mesh, scratch_types=scratch,
                  compiler_params=pltpu.CompilerParams(
                      use_tc_tiling_on_sc=False,
                      needs_layout_passes=False))
  return run(industries_t, funding_type, itblT, ftblT)


BT = 2048


def _mlp_body(ui_ref, af_ref,
              w1u_ref, w1i_ref, w1af_ref, b1_ref,
              w2_ref, b2_ref, w3_ref, b3_ref, out_ref):
  dot = functools.partial(jnp.dot, preferred_element_type=jnp.float32)
  u = ui_ref[:, :D_EMB]
  it = ui_ref[:, D_EMB:]
  x = (dot(u, w1u_ref[:]) + dot(it, w1i_ref[:])
       + lax.dot_general(af_ref[:], w1af_ref[:], (((0,), (0,)), ((), ())),
                         preferred_element_type=jnp.float32)
       + b1_ref[:])
  h1 = jnp.maximum(x, 0.0)
  h2 = jnp.maximum(dot(h1, w2_ref[:]) + b2_ref[:], 0.0)
  out_ref[:] = jnp.maximum(dot(h2, w3_ref[:]) + b3_ref[:], 0.0)


def _mlp(ui, afT, W1u, W1i, W1af, b1, W2, b2, W3, b3):
  h1, h2, d_out = W2.shape[0], W3.shape[0], W3.shape[1]
  grid = (B // BT,)
  full = lambda shape: pl.BlockSpec(shape, lambda i: (0, 0))
  return pl.pallas_call(
      _mlp_body,
      grid=grid,
      in_specs=[
          pl.BlockSpec((BT, 2 * D_EMB), lambda i: (i, 0)),
          pl.BlockSpec((AF_ROWS, BT), lambda i: (0, i)),
          full(W1u.shape), full(W1i.shape), full(W1af.shape),
          full((1, h1)),
          full(W2.shape), full((1, h2)),
          full(W3.shape), full((1, d_out)),
      ],
      out_specs=pl.BlockSpec((BT, d_out), lambda i: (i, 0)),
      out_shape=jax.ShapeDtypeStruct((B, d_out), jnp.float32),
  )(ui, afT, W1u, W1i, W1af, b1.reshape(1, -1),
    W2, b2.reshape(1, -1), W3, b3.reshape(1, -1))


def kernel(funds, startups, industries, funding_type, user_table, item_table,
           ind_table, ftype_table, W1, b1, W2, b2, W3, b3):
  funds = funds.astype(jnp.int32)
  startups = startups.astype(jnp.int32)
  funding_type = funding_type.astype(jnp.int32)
  industries_t = industries.astype(jnp.int32).T
  user2 = jnp.pad(user_table, ((0, 0), (0, D_EMB)))
  item2 = jnp.pad(item_table, ((0, 0), (D_EMB, 0)))
  itblT = ind_table.T.at[:, 0].set(0.0)
  ftblT = ftype_table.T
  ui = _sc_ui(funds, startups, user2, item2)
  afT = _sc_af(industries_t, funding_type, itblT, ftblT)
  W1u = W1[:D_EMB]
  W1i = W1[D_EMB:2 * D_EMB]
  W1af = jnp.concatenate([W1[2 * D_EMB + FTYPE_SZ:], W1[2 * D_EMB:2 * D_EMB + FTYPE_SZ]], axis=0)
  return _mlp(ui, afT, W1u, W1i, W1af, b1, W2, b2, W3, b3)


# speedup vs baseline: 1.1629x; 1.0253x over previous
import functools

import jax
import jax.numpy as jnp
from jax import lax
from jax.experimental import pallas as pl
from jax.experimental.pallas import tpu as pltpu
from jax.experimental.pallas import tpu_sc as plsc

B = 16384
D_EMB = 64
N_IND_SLOTS = 20
IND_SZ = 16
FTYPE_SZ = 4
N_INDS = 1000
N_FTYPES = 16
AF_ROWS = IND_SZ + FTYPE_SZ

NC = 2
NS = 16
L = 16
NW = NC * NS
BPW = B // NW
CHUNK = 128
NCHUNK = BPW // CHUNK
NGROUP = BPW // L


def _sc_emb_body(ids_h, table_h, out_h, *refs):
  idx_v = refs[0:NCHUNK]
  rows_v = refs[NCHUNK:2 * NCHUNK]
  sem = refs[2 * NCHUNK]
  wid = lax.axis_index("s") * NC + lax.axis_index("c")
  base = wid * BPW

  for k in range(NCHUNK):
    pltpu.sync_copy(ids_h.at[pl.ds(base + k * CHUNK, CHUNK)], idx_v[k])
  copies = [
      pltpu.async_copy(table_h.at[idx_v[k]], rows_v[k], sem)
      for k in range(NCHUNK)
  ]
  for k in range(NCHUNK):
    copies[k].wait()
    pltpu.sync_copy(rows_v[k], out_h.at[pl.ds(base + k * CHUNK, CHUNK)])


def _sc_emb(ids, table_pad):
  mesh = plsc.VectorSubcoreMesh(
      core_axis_name="c", subcore_axis_name="s",
      num_cores=NC, num_subcores=NS)
  f32 = jnp.float32
  scratch = [
      *[pltpu.VMEM((CHUNK,), jnp.int32) for _ in range(NCHUNK)],
      *[pltpu.VMEM((CHUNK, 2 * D_EMB), f32) for _ in range(NCHUNK)],
      pltpu.SemaphoreType.DMA,
  ]
  run = pl.kernel(_sc_emb_body,
                  out_type=jax.ShapeDtypeStruct((B, 2 * D_EMB), f32),
                  mesh=mesh, scratch_types=scratch,
                  compiler_params=pltpu.CompilerParams(
                      needs_layout_passes=False))
  return run(ids, table_pad)


def _sc_af_body(ind_t_h, ftype_h, itblT_h, ftblT_h, af_out,
                ftidx_v, indv_v, itblT_v, ftblT_v, af_stage):
  wid = lax.axis_index("s") * NC + lax.axis_index("c")
  base = wid * BPW

  pltpu.sync_copy(itblT_h, itblT_v)
  pltpu.sync_copy(ftblT_h, ftblT_v)
  pltpu.sync_copy(ftype_h.at[pl.ds(base, BPW)], ftidx_v)
  for j in range(N_IND_SLOTS):
    pltpu.sync_copy(ind_t_h.at[j, pl.ds(base, BPW)], indv_v.at[j])

  def group(g, carry):
    s = pl.multiple_of(g * L, L)
    idxs = [indv_v[j, pl.ds(s, L)] for j in range(N_IND_SLOTS)]
    cnt = jnp.full((L,), 0.0, jnp.float32)
    for j in range(N_IND_SLOTS):
      cnt = cnt + jnp.minimum(idxs[j], 1).astype(jnp.float32)
    inv = jnp.full((L,), 1.0, jnp.float32) / cnt
    for c in range(IND_SZ):
      colv = jnp.full((L,), c, jnp.int32)
      acc = plsc.load_gather(itblT_v, [colv, idxs[0]])
      for j in range(1, N_IND_SLOTS):
        acc = acc + plsc.load_gather(itblT_v, [colv, idxs[j]])
      af_stage[c, pl.ds(s, L)] = acc * inv
    ftv = ftidx_v[pl.ds(s, L)]
    for c in range(FTYPE_SZ):
      colv = jnp.full((L,), c, jnp.int32)
      af_stage[IND_SZ + c, pl.ds(s, L)] = plsc.load_gather(ftblT_v, [colv, ftv])
    return carry

  lax.fori_loop(0, NGROUP, group, 0)
  pltpu.sync_copy(af_stage, af_out.at[:, pl.ds(base, BPW)])


def _sc_af(industries_t, funding_type, itblT, ftblT):
  mesh = plsc.VectorSubcoreMesh(
      core_axis_name="c", subcore_axis_name="s",
      num_cores=NC, num_subcores=NS)
  f32 = jnp.float32
  scratch = [
      pltpu.VMEM((BPW,), jnp.int32),
      pltpu.VMEM((N_IND_SLOTS, BPW), jnp.int32),
      pltpu.VMEM((IND_SZ, N_INDS), f32),
      pltpu.VMEM((FTYPE_SZ, N_FTYPES), f32),
      pltpu.VMEM((AF_ROWS, BPW), f32),
  ]
  run = pl.kernel(_sc_af_body,
                  out_type=jax.ShapeDtypeStruct((AF_ROWS, B), f32),
                  mesh=mesh, scratch_types=scratch,
                  compiler_params=pltpu.CompilerParams(
                      use_tc_tiling_on_sc=False,
                      needs_layout_passes=False))
  return run(industries_t, funding_type, itblT, ftblT)


BT = 2048


def _mlp_body(u_ref, i_ref, af_ref,
              w1u_ref, w1i_ref, w1af_ref, b1_ref,
              w2_ref, b2_ref, w3_ref, b3_ref, out_ref):
  dot = functools.partial(jnp.dot, preferred_element_type=jnp.float32)
  x = (dot(u_ref[:, :D_EMB], w1u_ref[:]) + dot(i_ref[:, D_EMB:], w1i_ref[:])
       + lax.dot_general(af_ref[:], w1af_ref[:], (((0,), (0,)), ((), ())),
                         preferred_element_type=jnp.float32)
       + b1_ref[:])
  h1 = jnp.maximum(x, 0.0)
  h2 = jnp.maximum(dot(h1, w2_ref[:]) + b2_ref[:], 0.0)
  out_ref[:] = jnp.maximum(dot(h2, w3_ref[:]) + b3_ref[:], 0.0)


def _mlp(u128, i128, afT, W1u, W1i, W1af, b1, W2, b2, W3, b3):
  h1, h2, d_out = W2.shape[0], W3.shape[0], W3.shape[1]
  grid = (B // BT,)
  full = lambda shape: pl.BlockSpec(shape, lambda i: (0, 0))
  return pl.pallas_call(
      _mlp_body,
      grid=grid,
      in_specs=[
          pl.BlockSpec((BT, 2 * D_EMB), lambda i: (i, 0)),
          pl.BlockSpec((BT, 2 * D_EMB), lambda i: (i, 0)),
          pl.BlockSpec((AF_ROWS, BT), lambda i: (0, i)),
          full(W1u.shape), full(W1i.shape), full(W1af.shape),
          full((1, h1)),
          full(W2.shape), full((1, h2)),
          full(W3.shape), full((1, d_out)),
      ],
      out_specs=pl.BlockSpec((BT, d_out), lambda i: (i, 0)),
      out_shape=jax.ShapeDtypeStruct((B, d_out), jnp.float32),
  )(u128, i128, afT, W1u, W1i, W1af, b1.reshape(1, -1),
    W2, b2.reshape(1, -1), W3, b3.reshape(1, -1))


def kernel(funds, startups, industries, funding_type, user_table, item_table,
           ind_table, ftype_table, W1, b1, W2, b2, W3, b3):
  funds = funds.astype(jnp.int32)
  startups = startups.astype(jnp.int32)
  funding_type = funding_type.astype(jnp.int32)
  industries_t = industries.astype(jnp.int32).T
  user2 = jnp.pad(user_table, ((0, 0), (0, D_EMB)))
  item2 = jnp.pad(item_table, ((0, 0), (D_EMB, 0)))
  itblT = ind_table.T.at[:, 0].set(0.0)
  ftblT = ftype_table.T
  i128 = _sc_emb(startups, item2)
  afT = _sc_af(industries_t, funding_type, itblT, ftblT)
  u128 = _sc_emb(funds, user2)
  W1u = W1[:D_EMB]
  W1i = W1[D_EMB:2 * D_EMB]
  W1af = jnp.concatenate([W1[2 * D_EMB + FTYPE_SZ:], W1[2 * D_EMB:2 * D_EMB + FTYPE_SZ]], axis=0)
  return _mlp(u128, i128, afT, W1u, W1i, W1af, b1, W2, b2, W3, b3)
